# baseline (speedup 1.0000x reference)
import functools

import jax
import jax.numpy as jnp
from jax import lax
from jax.experimental import pallas as pl
from jax.experimental.pallas import tpu as pltpu
from jax.experimental.pallas import tpu_sc as plsc

VOCAB = 49408
MAX_POS = 77
EMBED_DIM = 768
BATCH = 4096
SEQ = 77

N_ROWS = BATCH * SEQ
NUM_CORES = 2
NUM_SUBCORES = 16
NUM_WORKERS = NUM_CORES * NUM_SUBCORES
ROWS_PER_WORKER = N_ROWS // NUM_WORKERS
CHUNK = 56
NUM_CHUNKS = ROWS_PER_WORKER // CHUNK
LANES = 16
D_VECS = EMBED_DIM // LANES


def _body(ids_hbm, tok_hbm, pos_hbm, out_hbm, idx_c, rows_v, pos_v, sem):
    wid = lax.axis_index("s") * NUM_CORES + lax.axis_index("c")
    base = wid * ROWS_PER_WORKER

    pltpu.sync_copy(pos_hbm, pos_v)

    def chunk_body(c, _):
        start = base + c * CHUNK
        pltpu.sync_copy(ids_hbm.at[pl.ds(start, CHUNK)], idx_c)
        pltpu.async_copy(tok_hbm.at[idx_c], rows_v, sem).wait()

        def row_body(i, _):
            p = lax.rem(c * CHUNK + i, MAX_POS)
            for j in range(D_VECS):
                sl = pl.ds(j * LANES, LANES)
                plsc.addupdate(rows_v.at[i, sl], pos_v[p, sl])
            return 0

        lax.fori_loop(0, CHUNK, row_body, 0)

        pltpu.sync_copy(rows_v, out_hbm.at[pl.ds(start, CHUNK)])
        return 0

    lax.fori_loop(0, NUM_CHUNKS, chunk_body, 0)


@jax.jit
def kernel(input_ids, token_table, position_table):
    ids_flat = input_ids.reshape(N_ROWS).astype(jnp.int32)
    mesh = plsc.VectorSubcoreMesh(core_axis_name="c", subcore_axis_name="s")
    out = pl.kernel(
        _body,
        out_type=jax.ShapeDtypeStruct((N_ROWS, EMBED_DIM), jnp.float32),
        mesh=mesh,
        scratch_types=[
            pltpu.VMEM((CHUNK,), jnp.int32),
            pltpu.VMEM((CHUNK, EMBED_DIM), jnp.float32),
            pltpu.VMEM((MAX_POS, EMBED_DIM), jnp.float32),
            pltpu.SemaphoreType.DMA,
        ],
    )(ids_flat, token_table, position_table)
    return out.reshape(BATCH, SEQ, EMBED_DIM)


# speedup vs baseline: 1.1509x; 1.1509x over previous
import jax
import jax.numpy as jnp
from jax import lax
from jax.experimental import pallas as pl
from jax.experimental.pallas import tpu as pltpu
from jax.experimental.pallas import tpu_sc as plsc

VOCAB = 49408
MAX_POS = 77
EMBED_DIM = 768
BATCH = 4096
SEQ = 77

N_ROWS = BATCH * SEQ
NUM_CORES = 2
NUM_SUBCORES = 16
NUM_WORKERS = NUM_CORES * NUM_SUBCORES
ROWS_PER_WORKER = N_ROWS // NUM_WORKERS
CHUNK = 32
NUM_CHUNKS = ROWS_PER_WORKER // CHUNK
LANES = 16
D_VECS = EMBED_DIM // LANES


def _body(ids_hbm, tok_hbm, pos_hbm, out_hbm,
          idx_all, rows0, rows1, pos_v, sg0, sg1, ss0, ss1):
    wid = lax.axis_index("s") * NUM_CORES + lax.axis_index("c")
    base = wid * ROWS_PER_WORKER

    rows = (rows0, rows1)
    sg = (sg0, sg1)
    ss = (ss0, ss1)

    pltpu.sync_copy(ids_hbm.at[pl.ds(base, ROWS_PER_WORKER)], idx_all)
    pltpu.sync_copy(pos_hbm, pos_v)

    def gather_start(c, b):
        pltpu.async_copy(
            tok_hbm.at[idx_all.at[pl.ds(c * CHUNK, CHUNK)]], rows[b], sg[b])

    gather_start(0, 0)

    def pair_body(t, _):
        for b in range(2):
            c = 2 * t + b
            pltpu.make_async_copy(tok_hbm.at[idx_all.at[pl.ds(0, CHUNK)]],
                                  rows[b], sg[b]).wait()

            @pl.when(c > 0)
            def _():
                pltpu.make_async_copy(
                    rows[1 - b], out_hbm.at[pl.ds(base, CHUNK)],
                    ss[1 - b]).wait()

            @pl.when(c + 1 < NUM_CHUNKS)
            def _():
                pltpu.async_copy(
                    tok_hbm.at[idx_all.at[pl.ds((c + 1) * CHUNK, CHUNK)]],
                    rows[1 - b], sg[1 - b])

            def row_body(i, _):
                p = lax.rem(c * CHUNK + i, MAX_POS)
                for j in range(D_VECS):
                    sl = pl.ds(j * LANES, LANES)
                    plsc.addupdate(rows[b].at[i, sl], pos_v[p, sl])
                return 0

            lax.fori_loop(0, CHUNK, row_body, 0)

            pltpu.async_copy(rows[b], out_hbm.at[pl.ds(base + c * CHUNK, CHUNK)],
                             ss[b])
        return 0

    lax.fori_loop(0, NUM_CHUNKS // 2, pair_body, 0)

    pltpu.make_async_copy(rows[1], out_hbm.at[pl.ds(base, CHUNK)], ss[1]).wait()


@jax.jit
def kernel(input_ids, token_table, position_table):
    ids_flat = input_ids.reshape(N_ROWS).astype(jnp.int32)
    mesh = plsc.VectorSubcoreMesh(core_axis_name="c", subcore_axis_name="s")
    out = pl.kernel(
        _body,
        out_type=jax.ShapeDtypeStruct((N_ROWS, EMBED_DIM), jnp.float32),
        mesh=mesh,
        scratch_types=[
            pltpu.VMEM((ROWS_PER_WORKER,), jnp.int32),
            pltpu.VMEM((CHUNK, EMBED_DIM), jnp.float32),
            pltpu.VMEM((CHUNK, EMBED_DIM), jnp.float32),
            pltpu.VMEM((MAX_POS, EMBED_DIM), jnp.float32),
            pltpu.SemaphoreType.DMA,
            pltpu.SemaphoreType.DMA,
            pltpu.SemaphoreType.DMA,
            pltpu.SemaphoreType.DMA,
        ],
    )(ids_flat, token_table, position_table)
    return out.reshape(BATCH, SEQ, EMBED_DIM)


# speedup vs baseline: 1.6431x; 1.4277x over previous
import jax
import jax.numpy as jnp
from jax import lax
from jax.experimental import pallas as pl
from jax.experimental.pallas import tpu as pltpu
from jax.experimental.pallas import tpu_sc as plsc

VOCAB = 49408
MAX_POS = 77
EMBED_DIM = 768
BATCH = 4096
SEQ = 77
SEQ_PAD = 80
HALF = SEQ_PAD // 2

NUM_CORES = 2
NUM_SUBCORES = 16
NUM_WORKERS = NUM_CORES * NUM_SUBCORES
SEQS_PER_WORKER = BATCH // NUM_WORKERS
IDX_PER_WORKER = SEQS_PER_WORKER * SEQ_PAD
NUM_CHUNKS = 2 * SEQS_PER_WORKER
GROUP = 16
LANES = 16
D_VECS = EMBED_DIM // LANES


def _body(ids_hbm, tok_hbm, pos_hbm, out_hbm,
          idx_g, rows0, rows1, pos_v, sg0, sg1, ss0, ss1):
    wid = lax.axis_index("s") * NUM_CORES + lax.axis_index("c")
    q_base = wid * SEQS_PER_WORKER

    rows = (rows0, rows1)
    sg = (sg0, sg1)
    ss = (ss0, ss1)

    def stage_group(tq):
        pltpu.sync_copy(
            ids_hbm.at[pl.ds(wid * IDX_PER_WORKER + tq * SEQ_PAD,
                             GROUP * SEQ_PAD)], idx_g)

    pltpu.sync_copy(pos_hbm, pos_v)
    stage_group(0)

    def gather_start(tmod, h, b):
        idx = idx_g.at[pl.ds(tmod * SEQ_PAD + h * HALF, HALF)]
        pltpu.async_copy(tok_hbm.at[idx], rows[b], sg[b])

    gather_start(0, 0, 0)

    def seq_body(t, _):
        tmod = lax.rem(t, GROUP)
        for k in range(2):
            c = 2 * t + k
            b = k
            q = q_base + t

            pltpu.make_async_copy(
                tok_hbm.at[idx_g.at[pl.ds(0, HALF)]], rows[b], sg[b]).wait()

            @pl.when(c > 0)
            def _():
                pltpu.make_async_copy(
                    rows[1 - b], out_hbm.at[0, pl.ds(0, HALF), :],
                    ss[1 - b]).wait()

            @pl.when(c + 1 < NUM_CHUNKS)
            def _():
                if k == 0:
                    gather_start(tmod, 1, 1)
                else:
                    @pl.when(tmod == GROUP - 1)
                    def _():
                        stage_group(t + 1)
                    gather_start(lax.rem(t + 1, GROUP), 0, 0)

            def row_body(i, _):
                vals = [pos_v[k * HALF + i, pl.ds(j * LANES, LANES)]
                        for j in range(D_VECS)]
                for j in range(D_VECS):
                    plsc.addupdate(rows[b].at[i, pl.ds(j * LANES, LANES)],
                                   vals[j])
                return 0

            lax.fori_loop(0, HALF, row_body, 0)

            pltpu.async_copy(rows[b], out_hbm.at[q, pl.ds(k * HALF, HALF), :],
                             ss[b])
        return 0

    lax.fori_loop(0, SEQS_PER_WORKER, seq_body, 0)

    pltpu.make_async_copy(rows[1], out_hbm.at[0, pl.ds(0, HALF), :],
                          ss[1]).wait()


@jax.jit
def kernel(input_ids, token_table, position_table):
    ids_pad = jnp.pad(input_ids.astype(jnp.int32),
                      ((0, 0), (0, SEQ_PAD - SEQ))).reshape(BATCH * SEQ_PAD)
    pos_pad = jnp.pad(position_table, ((0, SEQ_PAD - SEQ), (0, 0)))
    mesh = plsc.VectorSubcoreMesh(core_axis_name="c", subcore_axis_name="s")
    out_pad = pl.kernel(
        _body,
        out_type=jax.ShapeDtypeStruct((BATCH, SEQ_PAD, EMBED_DIM),
                                      jnp.float32),
        mesh=mesh,
        scratch_types=[
            pltpu.VMEM((GROUP * SEQ_PAD,), jnp.int32),
            pltpu.VMEM((HALF, EMBED_DIM), jnp.float32),
            pltpu.VMEM((HALF, EMBED_DIM), jnp.float32),
            pltpu.VMEM((SEQ_PAD, EMBED_DIM), jnp.float32),
            pltpu.SemaphoreType.DMA,
            pltpu.SemaphoreType.DMA,
            pltpu.SemaphoreType.DMA,
            pltpu.SemaphoreType.DMA,
        ],
    )(ids_pad, token_table, pos_pad)
    return out_pad[:, :SEQ, :]
